# baseline (speedup 1.0000x reference)
import jax
import jax.numpy as jnp
from jax.experimental import pallas as pl


def _copy_body(pe_ref, o_ref):
    o_ref[...] = pe_ref[...]


def kernel(x, pos_emb):
    seq = x.shape[1]
    d = pos_emb.shape[1]
    block = 4096
    out = pl.pallas_call(
        _copy_body,
        grid=(seq // block,),
        in_specs=[pl.BlockSpec((block, d), lambda i: (i, 0))],
        out_specs=pl.BlockSpec((block, d), lambda i: (i, 0)),
        out_shape=jax.ShapeDtypeStruct((seq, d), jnp.float32),
    )(pos_emb)
    return out
